# baseline (speedup 1.0000x reference)
import jax
import jax.numpy as jnp
from jax import lax
from jax.experimental import pallas as pl
from jax.experimental.pallas import tpu as pltpu
from jax.experimental.pallas import tpu_sc as plsc

NC = 2
NS = 16
NW = NC * NS
LANES = 16

BATCH = 1024
T = 77
TA = 72
TB = 8
FIX = T - TA
D = 768
G = D // LANES
PB = BATCH // NW


def _body(tok_a_hbm, tok_b_hbm, pos_hbm, table_hbm, out_hbm,
          idx_a, idx_b, pos_v, buf, buf_b, sem_a, sem_b):
    c = lax.axis_index("c")
    s = lax.axis_index("s")
    wid = s * NC + c
    base = wid * PB

    pltpu.sync_copy(tok_a_hbm.at[pl.ds(base * TA, PB * TA)], idx_a)
    pltpu.sync_copy(tok_b_hbm.at[pl.ds(base * TB, PB * TB)], idx_b)
    pltpu.sync_copy(pos_hbm, pos_v)

    @pl.loop(0, PB)
    def _batch(j):
        cp_a = pltpu.async_copy(
            table_hbm.at[idx_a.at[pl.ds(j * TA, TA)]], buf.at[pl.ds(0, TA)], sem_a)
        cp_b = pltpu.async_copy(
            table_hbm.at[idx_b.at[pl.ds(j * TB, TB)]], buf_b, sem_b)
        cp_a.wait()
        cp_b.wait()

        for r in range(FIX):
            for g in range(G):
                buf[TA + r, pl.ds(g * LANES, LANES)] = (
                    buf_b[TB - FIX + r, pl.ds(g * LANES, LANES)])

        @pl.loop(0, T)
        def _row(r):
            off = r * D
            for g in range(G):
                x = pos_v[pl.ds(off + g * LANES, LANES)]
                plsc.addupdate(buf.at[r, pl.ds(g * LANES, LANES)], x)

        pltpu.sync_copy(buf, out_hbm.at[base + j])


@jax.jit
def _embed(tokens, token_table, position_embedding):
    tokens = tokens.astype(jnp.int32)
    tok_a = tokens[:, :TA].reshape(-1)
    tok_b = tokens[:, T - TB:].reshape(-1)
    pos_flat = position_embedding.reshape(-1)
    mesh = plsc.VectorSubcoreMesh(core_axis_name="c", subcore_axis_name="s")
    return pl.kernel(
        _body,
        out_type=jax.ShapeDtypeStruct((BATCH, T, D), jnp.float32),
        mesh=mesh,
        scratch_types=[
            pltpu.VMEM((BATCH * TA // NW,), jnp.int32),
            pltpu.VMEM((BATCH * TB // NW,), jnp.int32),
            pltpu.VMEM((T * D,), jnp.float32),
            pltpu.VMEM((T, D), jnp.float32),
            pltpu.VMEM((TB, D), jnp.float32),
            pltpu.SemaphoreType.DMA,
            pltpu.SemaphoreType.DMA,
        ],
    )(tok_a, tok_b, pos_flat, token_table)


def kernel(tokens, token_table, position_embedding):
    return _embed(tokens, token_table, position_embedding)


# speedup vs baseline: 1.3299x; 1.3299x over previous
import jax
import jax.numpy as jnp
from jax import lax
from jax.experimental import pallas as pl
from jax.experimental.pallas import tpu as pltpu
from jax.experimental.pallas import tpu_sc as plsc

NC = 2
NS = 16
NW = NC * NS
LANES = 16

BATCH = 1024
T = 77
TA = 72
TB = 8
FIX = T - TA
D = 768
NH = 3
CW = D // NH
CG = CW // LANES
PB = BATCH // NW


def _body(tok_a_hbm, tok_b_hbm, pos_hbm, table_hbm, out_hbm,
          idx_a, idx_b, pos_v,
          buf0, buf1, buf2, fb0, fb1, fb2,
          gsem0, gsem1, gsem2, wsem0, wsem1, wsem2):
    bufs = (buf0, buf1, buf2)
    fbs = (fb0, fb1, fb2)
    gsems = (gsem0, gsem1, gsem2)
    wsems = (wsem0, wsem1, wsem2)

    c = lax.axis_index("c")
    s = lax.axis_index("s")
    wid = s * NC + c
    base = wid * PB

    pltpu.sync_copy(tok_a_hbm.at[pl.ds(base * TA, PB * TA)], idx_a)
    pltpu.sync_copy(tok_b_hbm.at[pl.ds(base * TB, PB * TB)], idx_b)
    pltpu.sync_copy(pos_hbm, pos_v)

    def start_gathers(j, h):
        col = pl.ds(h * CW, CW)
        pltpu.async_copy(
            table_hbm.at[idx_a.at[pl.ds(j * TA, TA)], col],
            bufs[h].at[pl.ds(0, TA)], gsems[h])
        pltpu.async_copy(
            table_hbm.at[idx_b.at[pl.ds(j * TB, TB)], col],
            fbs[h], gsems[h])

    def wait_gathers(j, h):
        col = pl.ds(h * CW, CW)
        pltpu.make_async_copy(
            table_hbm.at[idx_a.at[pl.ds(j * TA, TA)], col],
            bufs[h].at[pl.ds(0, TA)], gsems[h]).wait()
        pltpu.make_async_copy(
            table_hbm.at[idx_b.at[pl.ds(j * TB, TB)], col],
            fbs[h], gsems[h]).wait()

    def out_ref(j, h):
        return out_hbm.at[base + j].at[:, pl.ds(h * CW, CW)]

    def wait_write(j, h):
        pltpu.make_async_copy(bufs[h], out_ref(j, h), wsems[h]).wait()

    def process(j, h):
        wait_gathers(j, h)
        for r in range(FIX):
            for g in range(CG):
                x = fbs[h][TB - FIX + r, pl.ds(g * LANES, LANES)]
                p = pos_v[pl.ds((TA + r) * D + h * CW + g * LANES, LANES)]
                bufs[h][TA + r, pl.ds(g * LANES, LANES)] = x + p

        @pl.loop(0, TA)
        def _row(r):
            off = r * D + h * CW
            for g in range(CG):
                p = pos_v[pl.ds(off + g * LANES, LANES)]
                plsc.addupdate(bufs[h].at[r, pl.ds(g * LANES, LANES)], p)

        pltpu.async_copy(bufs[h], out_ref(j, h), wsems[h])

    start_gathers(0, 0)

    @pl.loop(0, PB)
    def _batch(j):
        @pl.when(j > 0)
        def _():
            wait_write(j - 1, 1)
        start_gathers(j, 1)
        process(j, 0)

        @pl.when(j > 0)
        def _():
            wait_write(j - 1, 2)
        start_gathers(j, 2)
        process(j, 1)

        @pl.when(j < PB - 1)
        def _():
            wait_write(j, 0)
            start_gathers(j + 1, 0)
        process(j, 2)

    for h in range(NH):
        wait_write(PB - 1, h)


@jax.jit
def _embed(tokens, token_table, position_embedding):
    tokens = tokens.astype(jnp.int32)
    tok_a = tokens[:, :TA].reshape(-1)
    tok_b = tokens[:, T - TB:].reshape(-1)
    pos_flat = position_embedding.reshape(-1)
    mesh = plsc.VectorSubcoreMesh(core_axis_name="c", subcore_axis_name="s")
    return pl.kernel(
        _body,
        out_type=jax.ShapeDtypeStruct((BATCH, T, D), jnp.float32),
        mesh=mesh,
        scratch_types=[
            pltpu.VMEM((BATCH * TA // NW,), jnp.int32),
            pltpu.VMEM((BATCH * TB // NW,), jnp.int32),
            pltpu.VMEM((T * D,), jnp.float32),
            pltpu.VMEM((T, CW), jnp.float32),
            pltpu.VMEM((T, CW), jnp.float32),
            pltpu.VMEM((T, CW), jnp.float32),
            pltpu.VMEM((TB, CW), jnp.float32),
            pltpu.VMEM((TB, CW), jnp.float32),
            pltpu.VMEM((TB, CW), jnp.float32),
            pltpu.SemaphoreType.DMA,
            pltpu.SemaphoreType.DMA,
            pltpu.SemaphoreType.DMA,
            pltpu.SemaphoreType.DMA,
            pltpu.SemaphoreType.DMA,
            pltpu.SemaphoreType.DMA,
        ],
    )(tok_a, tok_b, pos_flat, token_table)


def kernel(tokens, token_table, position_embedding):
    return _embed(tokens, token_table, position_embedding)
